# baseline (speedup 1.0000x reference)
import jax
import jax.numpy as jnp
from jax import lax
from jax.experimental import pallas as pl
from jax.experimental.pallas import tpu as pltpu

N = 32768
D = 128
K = 512
BN = 8192
NBLK = N // BN


def _body(a_ref, c_ref, idx_ref, loss_ref, cm_ref, acc_ref):
    @pl.when(pl.program_id(0) == 0)
    def _prep():
        c0 = c_ref[...]
        cm_ref[...] = -0.5 * jnp.sum(c0 * c0, axis=1, keepdims=True)
        acc_ref[0] = 0.0

    a = a_ref[...]
    u = lax.dot_general(c_ref[...], a, (((1,), (1,)), ((), ())),
                        preferred_element_type=jnp.float32) + cm_ref[...]
    m = jnp.max(u, axis=0, keepdims=True)
    row = lax.broadcasted_iota(jnp.int32, (K, BN), 0).astype(jnp.float32)
    idx = jnp.min(jnp.where(u >= m, row, float(K)), axis=0)
    idx_ref[...] = idx.astype(jnp.int32)
    acc_ref[0] += jnp.sum(a * a) - 2.0 * jnp.sum(m)

    @pl.when(pl.program_id(0) == NBLK - 1)
    def _fin():
        loss_ref[0] = acc_ref[0] * (1.0 / (N * D))


@jax.jit
def _run(embeddings, centroids):
    idx, loss = pl.pallas_call(
        _body,
        grid=(NBLK,),
        in_specs=[
            pl.BlockSpec((BN, D), lambda i: (i, 0)),
            pl.BlockSpec((K, D), lambda i: (0, 0)),
        ],
        out_specs=[
            pl.BlockSpec((BN,), lambda i: (i,)),
            pl.BlockSpec(memory_space=pltpu.SMEM),
        ],
        out_shape=[
            jax.ShapeDtypeStruct((N,), jnp.int32),
            jax.ShapeDtypeStruct((1,), jnp.float32),
        ],
        scratch_shapes=[
            pltpu.VMEM((K, 1), jnp.float32),
            pltpu.SMEM((1,), jnp.float32),
        ],
    )(embeddings, centroids)
    return idx, loss


import functools
from jax.experimental.pallas import tpu_sc as plsc

NW = 32
RPW = N // NW
CB = 128
NCH = RPW // CB

_sc_mesh = plsc.VectorSubcoreMesh(core_axis_name="c", subcore_axis_name="s")


@functools.partial(
    pl.kernel,
    out_type=jax.ShapeDtypeStruct((NW, 16), jnp.float32),
    mesh=_sc_mesh,
    scratch_types=[
        pltpu.VMEM((CB,), jnp.int32),
        pltpu.VMEM((CB, D), jnp.float32),
        pltpu.VMEM((CB, D), jnp.float32),
        pltpu.VMEM((16,), jnp.float32),
        pltpu.SemaphoreType.DMA,
    ],
)
def _sc_loss(emb_hbm, cent_hbm, idx_hbm, out_hbm, idx_v, e_v, c_v, acc_v, sem):
    wid = lax.axis_index("s") * 2 + lax.axis_index("c")
    base = wid * RPW
    accs = [jnp.zeros((16,), jnp.float32) for _ in range(8)]
    for chunk in range(NCH):
        row0 = base + chunk * CB
        pltpu.sync_copy(idx_hbm.at[pl.ds(row0, CB)], idx_v)
        pltpu.async_copy(emb_hbm.at[pl.ds(row0, CB)], e_v, sem).wait()
        pltpu.async_copy(cent_hbm.at[idx_v], c_v, sem).wait()

        def body(r, accs):
            out = []
            for j in range(8):
                e = e_v[r, pl.ds(16 * j, 16)]
                g = c_v[r, pl.ds(16 * j, 16)]
                d = e - g
                out.append(accs[j] + d * d)
            return tuple(out)

        accs = list(lax.fori_loop(0, CB, body, tuple(accs)))
    tot = accs[0]
    for j in range(1, 8):
        tot = tot + accs[j]
    acc_v[...] = tot
    pltpu.sync_copy(acc_v, out_hbm.at[wid])


def kernel(embeddings, centroids):
    idx, _ = _run(embeddings, centroids)
    part = _sc_loss(embeddings, centroids, idx)
    loss = jnp.sum(part) * (1.0 / (N * D))
    return (loss, idx)


# speedup vs baseline: 6.0838x; 6.0838x over previous
import jax
import jax.numpy as jnp
from jax import lax
from jax.experimental import pallas as pl
from jax.experimental.pallas import tpu as pltpu

N = 32768
D = 128
K = 512
BN = 8192
NBLK = N // BN


def _body(a_ref, c_ref, idx_ref, loss_ref, cm_ref, acc_ref):
    @pl.when(pl.program_id(0) == 0)
    def _prep():
        c0 = c_ref[...]
        cm_ref[...] = -0.5 * jnp.sum(c0 * c0, axis=1, keepdims=True)
        acc_ref[0] = 0.0

    a = a_ref[...]
    u = lax.dot_general(c_ref[...], a, (((1,), (1,)), ((), ())),
                        preferred_element_type=jnp.float32) + cm_ref[...]
    m = jnp.max(u, axis=0, keepdims=True)
    row = lax.broadcasted_iota(jnp.int32, (K, BN), 0).astype(jnp.float32)
    idx = jnp.min(jnp.where(u >= m, row, float(K)), axis=0)
    idx_ref[...] = idx.astype(jnp.int32)
    acc_ref[0] += jnp.sum(a * a) - 2.0 * jnp.sum(m)

    @pl.when(pl.program_id(0) == NBLK - 1)
    def _fin():
        loss_ref[0] = acc_ref[0] * (1.0 / (N * D))


@jax.jit
def _run(embeddings, centroids):
    idx, loss = pl.pallas_call(
        _body,
        grid=(NBLK,),
        in_specs=[
            pl.BlockSpec((BN, D), lambda i: (i, 0)),
            pl.BlockSpec((K, D), lambda i: (0, 0)),
        ],
        out_specs=[
            pl.BlockSpec((BN,), lambda i: (i,)),
            pl.BlockSpec(memory_space=pltpu.SMEM),
        ],
        out_shape=[
            jax.ShapeDtypeStruct((N,), jnp.int32),
            jax.ShapeDtypeStruct((1,), jnp.float32),
        ],
        scratch_shapes=[
            pltpu.VMEM((K, 1), jnp.float32),
            pltpu.SMEM((1,), jnp.float32),
        ],
    )(embeddings, centroids)
    return idx, loss


def kernel(embeddings, centroids):
    idx, loss = _run(embeddings, centroids)
    return (loss.reshape(()), idx)
